# baseline (device time: 80768 ns/iter reference)
import jax
import jax.numpy as jnp
from jax import lax
from jax.experimental import pallas as pl
from jax.experimental.pallas import tpu as pltpu

N_DEV = 16
SQ = 1024
D_MODEL = 1024
HQ_LOC = 8
DH = 128
BLK = 64
NGRP = 4
CH = SQ // N_DEV
SCALE = 0.08838834764831843

RS_SIZES = (512, 256, 128, 64)
RS_REGIONS = (0, 512, 768, 896)
XOR_MASKS = (1, 3, 4, 8)
COLS = (slice(0, 512), slice(512, 1024))
ORDERS = ((0, 1, 2, 3), (1, 0, 3, 2))


def kernel(x, Wq, K_ext, V_ext, Wo):
    my = lax.axis_index("i")
    x2 = x[0].astype(jnp.bfloat16)
    wq = Wq.astype(jnp.bfloat16)
    wo = Wo.astype(jnp.bfloat16)
    k_loc = jnp.transpose(
        lax.dynamic_slice_in_dim(K_ext[0], my * HQ_LOC, HQ_LOC, axis=1),
        (1, 0, 2),
    ).astype(jnp.bfloat16)
    v_loc = jnp.transpose(
        lax.dynamic_slice_in_dim(V_ext[0], my * HQ_LOC, HQ_LOC, axis=1),
        (1, 0, 2),
    ).astype(jnp.bfloat16)

    def body(x_ref, wq_ref, k_ref, v_ref, wo_ref, out_ref,
             comm_ref, send_ref, send_sems, recv_sems):
        me = lax.axis_index("i")
        partners = [jnp.bitwise_xor(me, m) for m in XOR_MASKS]
        w = jnp.mod(me, 4)
        bits = [
            jnp.logical_or(w == 1, w == 2).astype(jnp.int32),
            (w >= 2).astype(jnp.int32),
            jnp.mod(me // 4, 2),
            me // 8,
        ]

        barrier_sem = pltpu.get_barrier_semaphore()
        for nbr in partners:
            pl.semaphore_signal(
                barrier_sem, inc=1,
                device_id=(nbr,), device_id_type=pl.DeviceIdType.MESH,
            )
        pl.semaphore_wait(barrier_sem, 4)

        for r in range(NGRP):
            rows = [BLK * (r + NGRP * k) for k in range(4)]
            x_r = jnp.concatenate(
                [x_ref[pl.ds(s, BLK), :] for s in rows], axis=0
            )
            q_r = jnp.dot(x_r, wq_ref[:, :],
                          preferred_element_type=jnp.float32)
            q_rb = q_r.astype(jnp.bfloat16)
            ctx = []
            for h in range(HQ_LOC):
                k_h = jnp.concatenate(
                    [k_ref[h, pl.ds(s, BLK), :] for s in rows], axis=0
                )
                v_h = jnp.concatenate(
                    [v_ref[h, pl.ds(s, BLK), :] for s in rows], axis=0
                )
                s_rh = jnp.dot(q_rb[:, h * DH:(h + 1) * DH], k_h.T,
                               preferred_element_type=jnp.float32) * SCALE
                m = jnp.max(s_rh, axis=1, keepdims=True)
                w = jnp.exp(s_rh - m)
                w = w / jnp.sum(w, axis=1, keepdims=True)
                ctx.append(jnp.dot(w.astype(jnp.bfloat16), v_h,
                                   preferred_element_type=jnp.float32))
            ctx_r = jnp.concatenate(ctx, axis=1).astype(jnp.bfloat16)
            p_r = jnp.dot(ctx_r, wo_ref[:, :],
                          preferred_element_type=jnp.float32)
            for k in range(4):
                out_ref[pl.ds(rows[k], BLK), :] = p_r[BLK * k:BLK * (k + 1), :]

        cur_off = [jnp.int32(0), jnp.int32(0)]
        for r in range(4):
            s = RS_SIZES[r]
            r0 = RS_REGIONS[r]
            rdmas = []
            for p in (0, 1):
                d = ORDERS[p][r]
                b = bits[d]
                send_off = cur_off[p] + (1 - b) * s
                cur_off[p] = cur_off[p] + b * s
                send_ref[pl.ds(r0, s), COLS[p]] = out_ref[
                    pl.ds(send_off, s), COLS[p]
                ].astype(jnp.bfloat16)
                rdma = pltpu.make_async_remote_copy(
                    src_ref=send_ref.at[pl.ds(r0, s), COLS[p]],
                    dst_ref=comm_ref.at[pl.ds(r0, s), COLS[p]],
                    send_sem=send_sems.at[2 * r + p],
                    recv_sem=recv_sems.at[2 * r + p],
                    device_id=(partners[d],),
                    device_id_type=pl.DeviceIdType.MESH,
                )
                rdma.start()
                rdmas.append(rdma)
            for rdma in rdmas:
                rdma.wait()
            for p in (0, 1):
                sl = pl.ds(cur_off[p], s)
                out_ref[sl, COLS[p]] = (
                    out_ref[sl, COLS[p]]
                    + comm_ref[r0:r0 + s, COLS[p]].astype(jnp.float32)
                )

        blk_off = cur_off
        for j in range(4):
            r = 3 - j
            s = RS_SIZES[r]
            r0 = 960 + RS_REGIONS[r]
            rdmas = []
            p_offs = []
            for p in (0, 1):
                d = ORDERS[p][r]
                b = bits[d]
                send_ref[pl.ds(r0, s), COLS[p]] = out_ref[
                    pl.ds(blk_off[p], s), COLS[p]
                ].astype(jnp.bfloat16)
                rdma = pltpu.make_async_remote_copy(
                    src_ref=send_ref.at[pl.ds(r0, s), COLS[p]],
                    dst_ref=comm_ref.at[pl.ds(r0, s), COLS[p]],
                    send_sem=send_sems.at[2 * (4 + j) + p],
                    recv_sem=recv_sems.at[2 * (4 + j) + p],
                    device_id=(partners[d],),
                    device_id_type=pl.DeviceIdType.MESH,
                )
                rdma.start()
                rdmas.append(rdma)
                p_offs.append(blk_off[p] + (1 - 2 * b) * s)
                blk_off[p] = blk_off[p] - b * s
            for rdma in rdmas:
                rdma.wait()
            for p in (0, 1):
                out_ref[pl.ds(p_offs[p], s), COLS[p]] = comm_ref[
                    r0:r0 + s, COLS[p]
                ].astype(jnp.float32)

    out = pl.pallas_call(
        body,
        out_shape=jax.ShapeDtypeStruct((SQ, D_MODEL), jnp.float32),
        in_specs=[pl.BlockSpec(memory_space=pltpu.VMEM)] * 5,
        out_specs=pl.BlockSpec(memory_space=pltpu.VMEM),
        scratch_shapes=[
            pltpu.VMEM((1920, D_MODEL), jnp.bfloat16),
            pltpu.VMEM((1920, D_MODEL), jnp.bfloat16),
            pltpu.SemaphoreType.DMA((16,)),
            pltpu.SemaphoreType.DMA((16,)),
        ],
        compiler_params=pltpu.CompilerParams(
            collective_id=0,
            vmem_limit_bytes=100 * 1024 * 1024,
        ),
    )(x2, Wq, k_loc, v_loc, Wo)
    return out[None, :, :]


# device time: 29310 ns/iter; 2.7556x vs baseline; 2.7556x over previous
import os

import jax
import jax.numpy as jnp
from jax import lax
from jax.experimental import pallas as pl
from jax.experimental.pallas import tpu as pltpu

N_DEV = 16
SQ = 1024
D_MODEL = 1024
HQ_LOC = 8
DH = 128
BLK = 64
NGRP = 4
CH = SQ // N_DEV
SCALE = 0.08838834764831843

RS_SIZES = (512, 256, 128, 64)
RS_REGIONS = (0, 512, 768, 896)
XOR_MASKS = (1, 3, 4, 8)
COLS = (slice(0, 512), slice(512, 1024))
ORDERS = ((0, 1, 2, 3), (1, 0, 3, 2))

DO_COMM = os.environ.get("KERNEL_NO_COMM") != "1"


def kernel(x, Wq, K_ext, V_ext, Wo):
    my = lax.axis_index("i")
    x2 = x[0]
    k_loc = jnp.transpose(
        lax.dynamic_slice_in_dim(K_ext[0], my * HQ_LOC, HQ_LOC, axis=1),
        (1, 0, 2),
    )
    v_loc = jnp.transpose(
        lax.dynamic_slice_in_dim(V_ext[0], my * HQ_LOC, HQ_LOC, axis=1),
        (1, 0, 2),
    )

    def body(x_ref, wq_ref, k_ref, v_ref, wo_ref, out_ref,
             comm_ref, send_ref, send_sems, recv_sems):
        me = lax.axis_index("i")
        partners = [jnp.bitwise_xor(me, m) for m in XOR_MASKS]
        w = jnp.mod(me, 4)
        bits = [
            jnp.logical_or(w == 1, w == 2).astype(jnp.int32),
            (w >= 2).astype(jnp.int32),
            jnp.mod(me // 4, 2),
            me // 8,
        ]

        if DO_COMM:
            barrier_sem = pltpu.get_barrier_semaphore()
            for nbr in partners:
                pl.semaphore_signal(
                    barrier_sem, inc=1,
                    device_id=(nbr,), device_id_type=pl.DeviceIdType.MESH,
                )
            pl.semaphore_wait(barrier_sem, 4)

        for r in range(NGRP):
            rows = [BLK * (r + NGRP * k) for k in range(4)]
            x_r = jnp.concatenate(
                [x_ref[pl.ds(s, BLK), :] for s in rows], axis=0
            )
            q_r = jnp.dot(x_r, wq_ref[:, :],
                          preferred_element_type=jnp.float32)
            ctx = []
            for h in range(HQ_LOC):
                k_h = jnp.concatenate(
                    [k_ref[h, pl.ds(s, BLK), :] for s in rows], axis=0
                )
                v_h = jnp.concatenate(
                    [v_ref[h, pl.ds(s, BLK), :] for s in rows], axis=0
                )
                s_rh = jnp.dot(q_r[:, h * DH:(h + 1) * DH], k_h.T,
                               preferred_element_type=jnp.float32) * SCALE
                m = jnp.max(s_rh, axis=1, keepdims=True)
                w = jnp.exp(s_rh - m)
                w = w / jnp.sum(w, axis=1, keepdims=True)
                ctx.append(jnp.dot(w, v_h,
                                   preferred_element_type=jnp.float32))
            ctx_r = jnp.concatenate(ctx, axis=1)
            p_r = jnp.dot(ctx_r, wo_ref[:, :],
                          preferred_element_type=jnp.float32)
            for k in range(4):
                out_ref[pl.ds(rows[k], BLK), :] = p_r[BLK * k:BLK * (k + 1), :]

        if not DO_COMM:
            return
        cur_off = [jnp.int32(0), jnp.int32(0)]
        for r in range(4):
            s = RS_SIZES[r]
            r0 = RS_REGIONS[r]
            rdmas = []
            for p in (0, 1):
                d = ORDERS[p][r]
                b = bits[d]
                send_off = cur_off[p] + (1 - b) * s
                cur_off[p] = cur_off[p] + b * s
                send_ref[pl.ds(r0, s), COLS[p]] = out_ref[
                    pl.ds(send_off, s), COLS[p]
                ].astype(jnp.bfloat16)
                rdma = pltpu.make_async_remote_copy(
                    src_ref=send_ref.at[pl.ds(r0, s), COLS[p]],
                    dst_ref=comm_ref.at[pl.ds(r0, s), COLS[p]],
                    send_sem=send_sems.at[2 * r + p],
                    recv_sem=recv_sems.at[2 * r + p],
                    device_id=(partners[d],),
                    device_id_type=pl.DeviceIdType.MESH,
                )
                rdma.start()
                rdmas.append(rdma)
            for rdma in rdmas:
                rdma.wait()
            for p in (0, 1):
                sl = pl.ds(cur_off[p], s)
                out_ref[sl, COLS[p]] = (
                    out_ref[sl, COLS[p]]
                    + comm_ref[r0:r0 + s, COLS[p]].astype(jnp.float32)
                )

        blk_off = cur_off
        for j in range(4):
            r = 3 - j
            s = RS_SIZES[r]
            r0 = 960 + RS_REGIONS[r]
            rdmas = []
            p_offs = []
            for p in (0, 1):
                d = ORDERS[p][r]
                b = bits[d]
                send_ref[pl.ds(r0, s), COLS[p]] = out_ref[
                    pl.ds(blk_off[p], s), COLS[p]
                ].astype(jnp.bfloat16)
                rdma = pltpu.make_async_remote_copy(
                    src_ref=send_ref.at[pl.ds(r0, s), COLS[p]],
                    dst_ref=comm_ref.at[pl.ds(r0, s), COLS[p]],
                    send_sem=send_sems.at[2 * (4 + j) + p],
                    recv_sem=recv_sems.at[2 * (4 + j) + p],
                    device_id=(partners[d],),
                    device_id_type=pl.DeviceIdType.MESH,
                )
                rdma.start()
                rdmas.append(rdma)
                p_offs.append(blk_off[p] + (1 - 2 * b) * s)
                blk_off[p] = blk_off[p] - b * s
            for rdma in rdmas:
                rdma.wait()
            for p in (0, 1):
                out_ref[pl.ds(p_offs[p], s), COLS[p]] = comm_ref[
                    r0:r0 + s, COLS[p]
                ].astype(jnp.float32)

    out = pl.pallas_call(
        body,
        out_shape=jax.ShapeDtypeStruct((SQ, D_MODEL), jnp.float32),
        in_specs=[pl.BlockSpec(memory_space=pltpu.VMEM)] * 5,
        out_specs=pl.BlockSpec(memory_space=pltpu.VMEM),
        scratch_shapes=[
            pltpu.VMEM((1920, D_MODEL), jnp.bfloat16),
            pltpu.VMEM((1920, D_MODEL), jnp.bfloat16),
            pltpu.SemaphoreType.DMA((16,)),
            pltpu.SemaphoreType.DMA((16,)),
        ],
        compiler_params=pltpu.CompilerParams(
            collective_id=0 if DO_COMM else None,
            vmem_limit_bytes=100 * 1024 * 1024,
        ),
    )(x2, Wq, k_loc, v_loc, Wo)
    return out[None, :, :]
